# baseline (speedup 1.0000x reference)
import functools

import jax
import jax.numpy as jnp
from jax import lax
from jax.experimental import pallas as pl
from jax.experimental.pallas import tpu as pltpu
from jax.experimental.pallas import tpu_sc as plsc

N = 2097152
TABLE_SIZE = 123
TABLE_PAD = 128

_info = plsc.get_sparse_core_info()
_NC, _NS, _L = _info.num_cores, _info.num_subcores, _info.num_lanes
_NW = _NC * _NS
N_PER_W = N // _NW
CHUNK = 8192
N_CHUNKS = N_PER_W // CHUNK
UNROLL = 8


def _make_sc_kernel():
    mesh = plsc.VectorSubcoreMesh(core_axis_name="c", subcore_axis_name="s")

    @functools.partial(
        pl.kernel,
        mesh=mesh,
        out_type=jax.ShapeDtypeStruct((N,), jnp.float32),
        compiler_params=pltpu.CompilerParams(
            needs_layout_passes=False,
            disable_bounds_checks=True,
            disable_semaphore_checks=True,
            skip_device_barrier=True,
            use_tc_tiling_on_sc=False,
        ),
        scratch_types=[
            pltpu.VMEM((TABLE_PAD,), jnp.float32),
            pltpu.VMEM((2, CHUNK), jnp.int32),
            pltpu.VMEM((2, CHUNK), jnp.float32),
            pltpu.SemaphoreType.DMA,
            pltpu.SemaphoreType.DMA,
            pltpu.SemaphoreType.DMA,
            pltpu.SemaphoreType.DMA,
        ],
    )
    def gather_kernel(
        idx_hbm, table_hbm, out_hbm, table_v, idx_v, out_v, is0, is1, os0, os1
    ):
        wid = lax.axis_index("s") * _NC + lax.axis_index("c")
        base = wid * N_PER_W
        pltpu.sync_copy(table_hbm, table_v.at[pl.ds(0, TABLE_SIZE)])
        isems = (is0, is1)
        osems = (os0, os1)
        in_h = [None, None]
        out_h = [None, None]
        in_h[0] = pltpu.async_copy(
            idx_hbm.at[pl.ds(base, CHUNK)], idx_v.at[0], isems[0]
        )
        for ci in range(N_CHUNKS):
            b = ci % 2
            nb = 1 - b
            if ci + 1 < N_CHUNKS:
                in_h[nb] = pltpu.async_copy(
                    idx_hbm.at[pl.ds(base + (ci + 1) * CHUNK, CHUNK)],
                    idx_v.at[nb],
                    isems[nb],
                )
            in_h[b].wait()
            if out_h[b] is not None:
                out_h[b].wait()

            @plsc.parallel_loop(0, CHUNK // _L, unroll=UNROLL)
            def _body(i, _b=b):
                o = i * _L
                idx = idx_v[_b, pl.ds(o, _L)]
                out_v[_b, pl.ds(o, _L)] = plsc.load_gather(table_v, [idx])

            out_h[b] = pltpu.async_copy(
                out_v.at[b], out_hbm.at[pl.ds(base + ci * CHUNK, CHUNK)], osems[b]
            )
        out_h[0].wait()
        out_h[1].wait()

    return gather_kernel


_sc_kernel = _make_sc_kernel()


def kernel(atomic_numbers, e0s_tensor):
    return _sc_kernel(atomic_numbers.astype(jnp.int32), e0s_tensor)


# speedup vs baseline: 1.0172x; 1.0172x over previous
import functools

import jax
import jax.numpy as jnp
from jax import lax
from jax.experimental import pallas as pl
from jax.experimental.pallas import tpu as pltpu
from jax.experimental.pallas import tpu_sc as plsc

N = 2097152
TABLE_SIZE = 123
TABLE_PAD = 128

_info = plsc.get_sparse_core_info()
_NC, _NS, _L = _info.num_cores, _info.num_subcores, _info.num_lanes
_NW = _NC * _NS
N_PER_W = N // _NW
CHUNK = 16384
N_CHUNKS = N_PER_W // CHUNK
UNROLL = 4


def _make_sc_kernel():
    mesh = plsc.VectorSubcoreMesh(core_axis_name="c", subcore_axis_name="s")

    @functools.partial(
        pl.kernel,
        mesh=mesh,
        out_type=jax.ShapeDtypeStruct((N,), jnp.float32),
        compiler_params=pltpu.CompilerParams(
            needs_layout_passes=False,
            disable_bounds_checks=True,
            disable_semaphore_checks=True,
            skip_device_barrier=True,
            use_tc_tiling_on_sc=False,
        ),
        scratch_types=[
            pltpu.VMEM((TABLE_PAD,), jnp.float32),
            pltpu.VMEM((2, CHUNK), jnp.int32),
            pltpu.VMEM((2, CHUNK), jnp.float32),
            pltpu.SemaphoreType.DMA,
            pltpu.SemaphoreType.DMA,
            pltpu.SemaphoreType.DMA,
            pltpu.SemaphoreType.DMA,
        ],
    )
    def gather_kernel(
        idx_hbm, table_hbm, out_hbm, table_v, idx_v, out_v, is0, is1, os0, os1
    ):
        wid = lax.axis_index("s") * _NC + lax.axis_index("c")
        base = wid * N_PER_W
        pltpu.sync_copy(table_hbm, table_v.at[pl.ds(0, TABLE_SIZE)])
        isems = (is0, is1)
        osems = (os0, os1)
        in_h = [None, None]
        out_h = [None, None]
        in_h[0] = pltpu.async_copy(
            idx_hbm.at[pl.ds(base, CHUNK)], idx_v.at[0], isems[0]
        )
        for ci in range(N_CHUNKS):
            b = ci % 2
            nb = 1 - b
            if ci + 1 < N_CHUNKS:
                in_h[nb] = pltpu.async_copy(
                    idx_hbm.at[pl.ds(base + (ci + 1) * CHUNK, CHUNK)],
                    idx_v.at[nb],
                    isems[nb],
                )
            in_h[b].wait()
            if out_h[b] is not None:
                out_h[b].wait()

            @plsc.parallel_loop(0, CHUNK // _L, unroll=UNROLL)
            def _body(i, _b=b):
                o = i * _L
                idx = idx_v[_b, pl.ds(o, _L)]
                out_v[_b, pl.ds(o, _L)] = plsc.load_gather(table_v, [idx])

            out_h[b] = pltpu.async_copy(
                out_v.at[b], out_hbm.at[pl.ds(base + ci * CHUNK, CHUNK)], osems[b]
            )
        out_h[0].wait()
        out_h[1].wait()

    return gather_kernel


_sc_kernel = _make_sc_kernel()


def kernel(atomic_numbers, e0s_tensor):
    return _sc_kernel(atomic_numbers.astype(jnp.int32), e0s_tensor)


# speedup vs baseline: 1.0271x; 1.0098x over previous
import functools

import jax
import jax.numpy as jnp
from jax import lax
from jax.experimental import pallas as pl
from jax.experimental.pallas import tpu as pltpu
from jax.experimental.pallas import tpu_sc as plsc

N = 2097152
TABLE_SIZE = 123
TABLE_PAD = 128

_info = plsc.get_sparse_core_info()
_NC, _NS, _L = _info.num_cores, _info.num_subcores, _info.num_lanes
_NW = _NC * _NS
N_PER_W = N // _NW
CHUNK = 16384
N_CHUNKS = N_PER_W // CHUNK
UNROLL = 8


def _make_sc_kernel():
    mesh = plsc.VectorSubcoreMesh(core_axis_name="c", subcore_axis_name="s")

    @functools.partial(
        pl.kernel,
        mesh=mesh,
        out_type=jax.ShapeDtypeStruct((N,), jnp.float32),
        compiler_params=pltpu.CompilerParams(
            needs_layout_passes=False,
            disable_bounds_checks=True,
            disable_semaphore_checks=True,
            skip_device_barrier=True,
            use_tc_tiling_on_sc=False,
        ),
        scratch_types=[
            pltpu.VMEM((TABLE_PAD,), jnp.float32),
            pltpu.VMEM((2, CHUNK), jnp.int32),
            pltpu.VMEM((2, CHUNK), jnp.float32),
            pltpu.SemaphoreType.DMA,
            pltpu.SemaphoreType.DMA,
            pltpu.SemaphoreType.DMA,
            pltpu.SemaphoreType.DMA,
        ],
    )
    def gather_kernel(
        idx_hbm, table_hbm, out_hbm, table_v, idx_v, out_v, is0, is1, os0, os1
    ):
        wid = lax.axis_index("s") * _NC + lax.axis_index("c")
        base = wid * N_PER_W
        pltpu.sync_copy(table_hbm, table_v.at[pl.ds(0, TABLE_SIZE)])
        isems = (is0, is1)
        osems = (os0, os1)
        in_h = [None, None]
        out_h = [None, None]
        in_h[0] = pltpu.async_copy(
            idx_hbm.at[pl.ds(base, CHUNK)], idx_v.at[0], isems[0]
        )
        for ci in range(N_CHUNKS):
            b = ci % 2
            nb = 1 - b
            if ci + 1 < N_CHUNKS:
                in_h[nb] = pltpu.async_copy(
                    idx_hbm.at[pl.ds(base + (ci + 1) * CHUNK, CHUNK)],
                    idx_v.at[nb],
                    isems[nb],
                )
            in_h[b].wait()
            if out_h[b] is not None:
                out_h[b].wait()

            @plsc.parallel_loop(0, CHUNK // _L, unroll=UNROLL)
            def _body(i, _b=b):
                o = i * _L
                idx = idx_v[_b, pl.ds(o, _L)]
                out_v[_b, pl.ds(o, _L)] = plsc.load_gather(table_v, [idx])

            out_h[b] = pltpu.async_copy(
                out_v.at[b], out_hbm.at[pl.ds(base + ci * CHUNK, CHUNK)], osems[b]
            )
        out_h[0].wait()
        out_h[1].wait()

    return gather_kernel


_sc_kernel = _make_sc_kernel()


def kernel(atomic_numbers, e0s_tensor):
    return _sc_kernel(atomic_numbers.astype(jnp.int32), e0s_tensor)


# speedup vs baseline: 1.0618x; 1.0338x over previous
import functools

import jax
import jax.numpy as jnp
from jax import lax
from jax.experimental import pallas as pl
from jax.experimental.pallas import tpu as pltpu
from jax.experimental.pallas import tpu_sc as plsc

N = 2097152
TABLE_SIZE = 123
TABLE_PAD = 128

_info = plsc.get_sparse_core_info()
_NC, _NS, _L = _info.num_cores, _info.num_subcores, _info.num_lanes
_NW = _NC * _NS
N_PER_W = N // _NW
CHUNK = 16384
N_CHUNKS = N_PER_W // CHUNK
UNROLL = 8


def _make_sc_kernel():
    mesh = plsc.VectorSubcoreMesh(core_axis_name="c", subcore_axis_name="s")

    @functools.partial(
        pl.kernel,
        mesh=mesh,
        out_type=jax.ShapeDtypeStruct((N,), jnp.float32),
        compiler_params=pltpu.CompilerParams(
            needs_layout_passes=False,
            disable_bounds_checks=True,
            disable_semaphore_checks=True,
            skip_device_barrier=True,
            use_tc_tiling_on_sc=False,
        ),
        scratch_types=[
            pltpu.VMEM((TABLE_PAD,), jnp.float32),
            pltpu.VMEM((2, CHUNK), jnp.int32),
            pltpu.VMEM((2, CHUNK), jnp.float32),
            pltpu.SemaphoreType.DMA,
            pltpu.SemaphoreType.DMA,
            pltpu.SemaphoreType.DMA,
            pltpu.SemaphoreType.DMA,
        ],
    )
    def gather_kernel(
        idx_hbm, table_hbm, out_hbm, table_v, idx_v, out_v, is0, is1, os0, os1
    ):
        wid = lax.axis_index("s") * _NC + lax.axis_index("c")
        base = wid * N_PER_W
        isems = (is0, is1)
        osems = (os0, os1)
        in_h = [None, None]
        out_h = [None, None]
        in_h[0] = pltpu.async_copy(
            idx_hbm.at[pl.ds(base, CHUNK)], idx_v.at[0], isems[0]
        )
        in_h[1] = pltpu.async_copy(
            idx_hbm.at[pl.ds(base + CHUNK, CHUNK)], idx_v.at[1], isems[1]
        )
        pltpu.sync_copy(table_hbm, table_v.at[pl.ds(0, TABLE_SIZE)])
        for ci in range(N_CHUNKS):
            b = ci % 2
            in_h[b].wait()
            if out_h[b] is not None:
                out_h[b].wait()

            @plsc.parallel_loop(0, CHUNK // _L, unroll=UNROLL)
            def _body(i, _b=b):
                o = i * _L
                idx = idx_v[_b, pl.ds(o, _L)]
                out_v[_b, pl.ds(o, _L)] = plsc.load_gather(table_v, [idx])

            if ci + 2 < N_CHUNKS:
                in_h[b] = pltpu.async_copy(
                    idx_hbm.at[pl.ds(base + (ci + 2) * CHUNK, CHUNK)],
                    idx_v.at[b],
                    isems[b],
                )
            out_h[b] = pltpu.async_copy(
                out_v.at[b], out_hbm.at[pl.ds(base + ci * CHUNK, CHUNK)], osems[b]
            )
        out_h[0].wait()
        out_h[1].wait()

    return gather_kernel


_sc_kernel = _make_sc_kernel()


def kernel(atomic_numbers, e0s_tensor):
    return _sc_kernel(atomic_numbers.astype(jnp.int32), e0s_tensor)


# speedup vs baseline: 1.0838x; 1.0207x over previous
import functools

import jax
import jax.numpy as jnp
from jax import lax
from jax.experimental import pallas as pl
from jax.experimental.pallas import tpu as pltpu
from jax.experimental.pallas import tpu_sc as plsc

N = 2097152
TABLE_SIZE = 123
TABLE_PAD = 128

_info = plsc.get_sparse_core_info()
_NC, _NS, _L = _info.num_cores, _info.num_subcores, _info.num_lanes
_NW = _NC * _NS
N_PER_W = N // _NW
CHUNK = 16384
SCHED = [8192, 16384, 16384, 16384, 8192]
UNROLL = 8


def _make_sc_kernel():
    mesh = plsc.VectorSubcoreMesh(core_axis_name="c", subcore_axis_name="s")

    @functools.partial(
        pl.kernel,
        mesh=mesh,
        out_type=jax.ShapeDtypeStruct((N,), jnp.float32),
        compiler_params=pltpu.CompilerParams(
            needs_layout_passes=False,
            disable_bounds_checks=True,
            disable_semaphore_checks=True,
            skip_device_barrier=True,
            use_tc_tiling_on_sc=False,
        ),
        scratch_types=[
            pltpu.VMEM((TABLE_PAD,), jnp.float32),
            pltpu.VMEM((2, CHUNK), jnp.int32),
            pltpu.VMEM((2, CHUNK), jnp.float32),
            pltpu.SemaphoreType.DMA,
            pltpu.SemaphoreType.DMA,
            pltpu.SemaphoreType.DMA,
            pltpu.SemaphoreType.DMA,
        ],
    )
    def gather_kernel(
        idx_hbm, table_hbm, out_hbm, table_v, idx_v, out_v, is0, is1, os0, os1
    ):
        wid = lax.axis_index("s") * _NC + lax.axis_index("c")
        base = wid * N_PER_W
        isems = (is0, is1)
        osems = (os0, os1)
        offs = [0]
        for sz in SCHED:
            offs.append(offs[-1] + sz)
        in_h = [None, None]
        out_h = [None, None]
        in_h[0] = pltpu.async_copy(
            idx_hbm.at[pl.ds(base + offs[0], SCHED[0])],
            idx_v.at[0, pl.ds(0, SCHED[0])],
            isems[0],
        )
        in_h[1] = pltpu.async_copy(
            idx_hbm.at[pl.ds(base + offs[1], SCHED[1])],
            idx_v.at[1, pl.ds(0, SCHED[1])],
            isems[1],
        )
        pltpu.sync_copy(table_hbm, table_v.at[pl.ds(0, TABLE_SIZE)])
        for ci, sz in enumerate(SCHED):
            b = ci % 2
            in_h[b].wait()
            if out_h[b] is not None:
                out_h[b].wait()

            @plsc.parallel_loop(0, sz // _L, unroll=UNROLL)
            def _body(i, _b=b):
                o = i * _L
                idx = idx_v[_b, pl.ds(o, _L)]
                out_v[_b, pl.ds(o, _L)] = plsc.load_gather(table_v, [idx])

            if ci + 2 < len(SCHED):
                in_h[b] = pltpu.async_copy(
                    idx_hbm.at[pl.ds(base + offs[ci + 2], SCHED[ci + 2])],
                    idx_v.at[b, pl.ds(0, SCHED[ci + 2])],
                    isems[b],
                )
            out_h[b] = pltpu.async_copy(
                out_v.at[b, pl.ds(0, sz)],
                out_hbm.at[pl.ds(base + offs[ci], sz)],
                osems[b],
            )
        out_h[0].wait()
        out_h[1].wait()

    return gather_kernel


_sc_kernel = _make_sc_kernel()


def kernel(atomic_numbers, e0s_tensor):
    return _sc_kernel(atomic_numbers.astype(jnp.int32), e0s_tensor)
